# baseline (speedup 1.0000x reference)
import functools
import jax
import jax.numpy as jnp
from jax import lax
from jax.experimental import pallas as pl
from jax.experimental.pallas import tpu as pltpu
from jax.experimental.pallas import tpu_sc as plsc

B_ROWS = 4096
B_COLS = 200
B_TOTAL = B_ROWS * B_COLS
D = 64
SCALE = 8.0

NUM_CORES = 2
NUM_SUBCORES = 16
NW = NUM_CORES * NUM_SUBCORES
PER_W = B_TOTAL // NW
CHUNK = 512
N_CHUNK = PER_W // CHUNK

_mesh = plsc.VectorSubcoreMesh(core_axis_name="c", subcore_axis_name="s")


@functools.partial(
    pl.kernel,
    mesh=_mesh,
    out_type=jax.ShapeDtypeStruct((B_TOTAL, D), jnp.float32),
    scratch_types=[
        pltpu.VMEM((CHUNK,), jnp.int32),
        pltpu.VMEM((CHUNK, D), jnp.float32),
        pltpu.SemaphoreType.DMA,
    ],
    compiler_params=pltpu.CompilerParams(use_tc_tiling_on_sc=False),
)
def _emb_lookup(idx_hbm, table_hbm, out_hbm, idx_v, rows_v, sem):
    wid = lax.axis_index("s") * NUM_CORES + lax.axis_index("c")
    base = wid * PER_W

    def chunk_body(c, carry):
        off = base + c * CHUNK
        pltpu.sync_copy(idx_hbm.at[pl.ds(off, CHUNK)], idx_v)
        pltpu.async_copy(table_hbm.at[idx_v], rows_v, sem).wait()

        def scale_row(r, carry2):
            for j in range(D // 16):
                sl = pl.ds(16 * j, 16)
                rows_v[r, sl] = rows_v[r, sl] * SCALE
            return carry2

        lax.fori_loop(0, CHUNK, scale_row, 0)
        pltpu.sync_copy(rows_v, out_hbm.at[pl.ds(off, CHUNK)])
        return carry

    lax.fori_loop(0, N_CHUNK, chunk_body, 0)


def kernel(x, table):
    idx = x.reshape(-1).astype(jnp.int32)
    out = _emb_lookup(idx, table)
    return out.reshape(B_ROWS, B_COLS, D)


# speedup vs baseline: 1.1347x; 1.1347x over previous
import functools
import jax
import jax.numpy as jnp
from jax import lax
from jax.experimental import pallas as pl
from jax.experimental.pallas import tpu as pltpu
from jax.experimental.pallas import tpu_sc as plsc

B_ROWS = 4096
B_COLS = 200
B_TOTAL = B_ROWS * B_COLS
D = 64
SCALE = 8.0

NUM_CORES = 2
NUM_SUBCORES = 16
NW = NUM_CORES * NUM_SUBCORES
PER_W = B_TOTAL // NW
CHUNK = 256
N_CHUNK = PER_W // CHUNK
NBUF = 4
ROW_UNROLL = 4

_mesh = plsc.VectorSubcoreMesh(core_axis_name="c", subcore_axis_name="s")


@functools.partial(
    pl.kernel,
    mesh=_mesh,
    out_type=jax.ShapeDtypeStruct((NW, N_CHUNK, CHUNK, D), jnp.float32),
    scratch_types=[
        pltpu.VMEM((N_CHUNK, CHUNK), jnp.int32),
        *[pltpu.VMEM((CHUNK, D), jnp.float32) for _ in range(NBUF)],
        *[pltpu.SemaphoreType.DMA for _ in range(2 * NBUF)],
    ],
    compiler_params=pltpu.CompilerParams(use_tc_tiling_on_sc=False),
)
def _emb_lookup(idx_hbm, table_hbm, out_hbm, idx_v, *bufs_and_sems):
    rows = bufs_and_sems[:NBUF]
    gsem = bufs_and_sems[NBUF:2 * NBUF]
    wsem = bufs_and_sems[2 * NBUF:]
    wid = lax.axis_index("s") * NUM_CORES + lax.axis_index("c")

    pltpu.sync_copy(idx_hbm.at[wid], idx_v)

    def start_gather(j, b):
        pltpu.make_async_copy(table_hbm.at[idx_v.at[j]], rows[b], gsem[b]).start()

    def wait_gather(j, b):
        pltpu.make_async_copy(table_hbm.at[idx_v.at[j]], rows[b], gsem[b]).wait()

    def start_write(j, b):
        pltpu.make_async_copy(rows[b], out_hbm.at[wid, j], wsem[b]).start()

    def wait_write(j, b):
        pltpu.make_async_copy(rows[b], out_hbm.at[wid, j], wsem[b]).wait()

    def scale_buf(b):
        def scale_rows(r, carry):
            r0 = r * ROW_UNROLL
            for dr in range(ROW_UNROLL):
                for jj in range(D // 16):
                    sl = pl.ds(16 * jj, 16)
                    rows[b][r0 + dr, sl] = rows[b][r0 + dr, sl] * SCALE
            return carry

        lax.fori_loop(0, CHUNK // ROW_UNROLL, scale_rows, 0)

    for b in range(NBUF):
        start_gather(b, b)

    def body(g, carry):
        j0 = g * NBUF
        for b in range(NBUF):
            wait_gather(j0 + b, b)
            scale_buf(b)
            start_write(j0 + b, b)
        for b in range(NBUF):
            wait_write(j0 + b, b)
            start_gather(j0 + NBUF + b, b)
        return carry

    lax.fori_loop(0, N_CHUNK // NBUF - 1, body, 0)

    jlast = N_CHUNK - NBUF
    for b in range(NBUF):
        wait_gather(jlast + b, b)
        scale_buf(b)
        start_write(jlast + b, b)
    for b in range(NBUF):
        wait_write(jlast + b, b)


def kernel(x, table):
    idx = x.reshape(NW, N_CHUNK, CHUNK).astype(jnp.int32)
    out = _emb_lookup(idx, table)
    return out.reshape(B_ROWS, B_COLS, D)
